# Initial kernel scaffold; baseline (speedup 1.0000x reference)
import jax
import jax.numpy as jnp
from jax.experimental import pallas as pl


def kernel(feat, edge_index, edge_weight, W1, gamma1, beta1, W2, gamma2, beta2, W3, b3):
    raise NotImplementedError("write your pallas kernel here")


# speedup vs baseline: 5.5240x; 5.5240x over previous
import functools
import math

import jax
import jax.numpy as jnp
from jax import lax
from jax.experimental import pallas as pl
from jax.experimental.pallas import tpu as pltpu
from jax.experimental.pallas import tpu_sc as plsc

N = 10000
E = 320000
F = 128
C = 128
NC = 40
D3 = 64
H = 64
EPS = 1e-05
BN_RS = 1.0 / math.sqrt(1.0 + EPS)

NCORES = 2
NSUB = 16
NW = NCORES * NSUB
K = 80
RPS = 624
TAIL = N - NSUB * RPS
ZR = 64


def _make_agg(D, col_split):
  nslab = NSUB if col_split else NW
  epw = E // nslab
  nchunk = epw // K
  mesh = plsc.VectorSubcoreMesh(core_axis_name="c", subcore_axis_name="s")

  def body(ht_hbm, pk_hbm, w_hbm, out_hbm,
           pk_v, src_v, dst_v, w_v, gbuf, zbuf, acc, gsem, ssem):
    cid = lax.axis_index("c")
    sid = lax.axis_index("s")
    slab = sid if col_split else sid * NCORES + cid

    def zrow(r, _):
      for j in range(D // 16):
        zbuf[r, pl.ds(16 * j, 16)] = jnp.zeros((16,), jnp.float32)
      return 0
    lax.fori_loop(0, ZR, zrow, 0)
    base = sid * RPS
    for b in range(RPS // ZR):
      pltpu.sync_copy(zbuf.at[pl.ds(0, ZR)], acc.at[pl.ds(base + b * ZR, ZR)])
    rem = RPS % ZR
    if rem:
      pltpu.sync_copy(zbuf.at[pl.ds(0, rem)],
                      acc.at[pl.ds(base + RPS - rem, rem)])

    @pl.when(sid == NSUB - 1)
    def _zero_tail():
      pltpu.sync_copy(zbuf.at[pl.ds(0, TAIL)],
                      acc.at[pl.ds(NSUB * RPS, TAIL)])

    pltpu.sync_copy(pk_hbm.at[slab], pk_v)
    pltpu.sync_copy(w_hbm.at[slab], w_v)
    soff = cid * N if col_split else 0

    def unpack(c, _):
      for g in range(K // 16):
        sl = pl.ds(16 * g, 16)
        v = pk_v[c, sl]
        src_v[c, sl] = lax.shift_right_logical(v, 16) + soff
        dst_v[c, sl] = lax.bitwise_and(v, jnp.int32(0xFFFF))
      return 0
    lax.fori_loop(0, nchunk, unpack, 0)
    plsc.subcore_barrier()

    def chunk(c, _):
      pltpu.async_copy(ht_hbm.at[src_v.at[c]], gbuf, gsem).wait()
      for g in range(K // 16):
        w16 = w_v[pl.ds(c * K + g * 16, 16)]
        for i in range(16):
          e = g * 16 + i
          wv = lax.gather(
              w16, jnp.full((16, 1), i, jnp.int32),
              dimension_numbers=lax.GatherDimensionNumbers(
                  offset_dims=(), collapsed_slice_dims=(0,),
                  start_index_map=(0,)),
              slice_sizes=(1,),
              mode=lax.GatherScatterMode.PROMISE_IN_BOUNDS)
          for j in range(D // 16):
            sl = pl.ds(16 * j, 16)
            gbuf[e, sl] = gbuf[e, sl] * wv
      pltpu.async_copy(gbuf, acc.at[dst_v.at[c]], ssem, add=True).wait()
      return 0
    lax.fori_loop(0, nchunk, chunk, 0)

    plsc.subcore_barrier()
    pltpu.sync_copy(acc.at[pl.ds(base, RPS)],
                    out_hbm.at[cid, pl.ds(base, RPS)])

    @pl.when(sid == NSUB - 1)
    def _copy_tail():
      pltpu.sync_copy(acc.at[pl.ds(NSUB * RPS, TAIL)],
                      out_hbm.at[cid, pl.ds(NSUB * RPS, TAIL)])

  return pl.kernel(
      body,
      out_type=jax.ShapeDtypeStruct((NCORES, N, D), jnp.float32),
      mesh=mesh,
      compiler_params=pltpu.CompilerParams(use_tc_tiling_on_sc=False),
      scratch_types=[
          pltpu.VMEM((nchunk, K), jnp.int32),
          pltpu.VMEM((nchunk, K), jnp.int32),
          pltpu.VMEM((nchunk, K), jnp.int32),
          pltpu.VMEM((epw,), jnp.float32),
          pltpu.VMEM((K, D), jnp.float32),
          pltpu.VMEM((ZR, D), jnp.float32),
          pltpu.VMEM_SHARED((N, D), jnp.float32),
          pltpu.SemaphoreType.DMA,
          pltpu.SemaphoreType.DMA,
      ],
  )


_agg_col = _make_agg(H, True)
_agg_edge = _make_agg(D3, False)

_R = 1000


def _mm_split_body(x_ref, w_ref, o_ref):
  res = jnp.dot(x_ref[...], w_ref[...], preferred_element_type=jnp.float32)
  o_ref[0] = res[:, :H]
  o_ref[1] = res[:, H:]


def _mm_split(x, w):
  n, f = x.shape
  return pl.pallas_call(
      _mm_split_body,
      grid=(n // _R,),
      in_specs=[pl.BlockSpec((_R, f), lambda i: (i, 0)),
                pl.BlockSpec((f, 2 * H), lambda i: (0, 0))],
      out_specs=pl.BlockSpec((2, _R, H), lambda i: (0, i, 0)),
      out_shape=jax.ShapeDtypeStruct((2, n, H), jnp.float32),
  )(x, w)


def _bn_relu_mm_body(split_out, p_ref, g_ref, b_ref, w_ref, o_ref):
  h0 = jnp.maximum(p_ref[0] * (g_ref[0] * BN_RS) + b_ref[0], 0.0)
  h1 = jnp.maximum(p_ref[1] * (g_ref[1] * BN_RS) + b_ref[1], 0.0)
  res = (jnp.dot(h0, w_ref[0], preferred_element_type=jnp.float32) +
         jnp.dot(h1, w_ref[1], preferred_element_type=jnp.float32))
  if split_out:
    o_ref[0] = res[:, :H]
    o_ref[1] = res[:, H:]
  else:
    o_ref[...] = res


def _bn_relu_mm(p, gamma, beta, w, split_out):
  n = p.shape[1]
  co = w.shape[1]
  g2 = gamma.reshape(2, H)
  b2 = beta.reshape(2, H)
  w3 = w.reshape(2, H, co)
  if split_out:
    out_specs = pl.BlockSpec((2, _R, H), lambda i: (0, i, 0))
    out_shape = jax.ShapeDtypeStruct((2, n, H), jnp.float32)
  else:
    out_specs = pl.BlockSpec((_R, co), lambda i: (i, 0))
    out_shape = jax.ShapeDtypeStruct((n, co), jnp.float32)
  return pl.pallas_call(
      functools.partial(_bn_relu_mm_body, split_out),
      grid=(n // _R,),
      in_specs=[pl.BlockSpec((2, _R, H), lambda i: (0, i, 0)),
                pl.BlockSpec((2, H), lambda i: (0, 0)),
                pl.BlockSpec((2, H), lambda i: (0, 0)),
                pl.BlockSpec((2, H, co), lambda i: (0, 0, 0))],
      out_specs=out_specs,
      out_shape=out_shape,
  )(p, g2, b2, w3)


def _bias_body(p0_ref, p1_ref, b_ref, o_ref):
  s = p0_ref[...] + p1_ref[...]
  o_ref[...] = s[:, :NC] + b_ref[...]


def _bias_sum(p0, p1, b3):
  n = p0.shape[0]
  b2 = b3.reshape(1, NC)
  return pl.pallas_call(
      _bias_body,
      grid=(n // _R,),
      in_specs=[pl.BlockSpec((_R, D3), lambda i: (i, 0)),
                pl.BlockSpec((_R, D3), lambda i: (i, 0)),
                pl.BlockSpec((1, NC), lambda i: (0, 0))],
      out_specs=pl.BlockSpec((_R, NC), lambda i: (i, 0)),
      out_shape=jax.ShapeDtypeStruct((n, NC), jnp.float32),
  )(p0, p1, b2)


@jax.jit
def kernel(feat, edge_index, edge_weight, W1, gamma1, beta1,
           W2, gamma2, beta2, W3, b3):
  pk = (edge_index[0] << 16) | edge_index[1]
  pk_c = pk.reshape(NSUB, E // NSUB // K, K)
  pk_e = pk.reshape(NW, E // NW // K, K)
  w_c = edge_weight.reshape(NSUB, E // NSUB)
  w_e = edge_weight.reshape(NW, E // NW)
  W3p = jnp.pad(W3, ((0, 0), (0, D3 - NC)))

  ht1 = _mm_split(feat, W1).reshape(2 * N, H)
  p1 = _agg_col(ht1, pk_c, w_c)
  ht2 = _bn_relu_mm(p1, gamma1, beta1, W2, True).reshape(2 * N, H)
  p2 = _agg_col(ht2, pk_c, w_c)
  ht3 = _bn_relu_mm(p2, gamma2, beta2, W3p, False)
  p3 = _agg_edge(ht3, pk_e, w_e)
  return _bias_sum(p3[0], p3[1], b3)


# speedup vs baseline: 9.7720x; 1.7690x over previous
import functools
import math

import jax
import jax.numpy as jnp
from jax import lax
from jax.experimental import pallas as pl
from jax.experimental.pallas import tpu as pltpu
from jax.experimental.pallas import tpu_sc as plsc

N = 10000
E = 320000
F = 128
C = 128
NC = 40
D3 = 64
H = 64
EPS = 1e-05
BN_RS = 1.0 / math.sqrt(1.0 + EPS)

NCORES = 2
NSUB = 16
NW = NCORES * NSUB
K = 80
NBUF = 2
RPS = 624
TAIL = N - NSUB * RPS
ZR = 64


def _make_agg(D, col_split):
  nslab = NSUB if col_split else NW
  epw = E // nslab
  nchunk = epw // K
  mesh = plsc.VectorSubcoreMesh(core_axis_name="c", subcore_axis_name="s")

  def body(ht_hbm, pk_hbm, w_hbm, out_hbm,
           pk_v, dst_v, w_v, gbuf, sbuf, zbuf, acc, gsems, ssems):
    cid = lax.axis_index("c")
    sid = lax.axis_index("s")
    slab = sid if col_split else sid * NCORES + cid

    def zrow(r, _):
      for j in range(D // 16):
        zbuf[r, pl.ds(16 * j, 16)] = jnp.zeros((16,), jnp.float32)
      return 0
    lax.fori_loop(0, ZR, zrow, 0)
    base = sid * RPS
    for b in range(RPS // ZR):
      pltpu.sync_copy(zbuf.at[pl.ds(0, ZR)], acc.at[pl.ds(base + b * ZR, ZR)])
    rem = RPS % ZR
    if rem:
      pltpu.sync_copy(zbuf.at[pl.ds(0, rem)],
                      acc.at[pl.ds(base + RPS - rem, rem)])

    @pl.when(sid == NSUB - 1)
    def _zero_tail():
      pltpu.sync_copy(zbuf.at[pl.ds(0, TAIL)],
                      acc.at[pl.ds(NSUB * RPS, TAIL)])

    pltpu.sync_copy(pk_hbm.at[slab], pk_v)
    pltpu.sync_copy(w_hbm.at[slab], w_v)
    soff = cid * N if col_split else 0

    def unpack(c, _):
      for g in range(K // 16):
        sl = pl.ds(16 * g, 16)
        v = pk_v[c, sl]
        dst_v[c, sl] = lax.bitwise_and(v, jnp.int32(0xFFFF))
        pk_v[c, sl] = lax.shift_right_logical(v, 16) + soff
      return 0
    lax.fori_loop(0, nchunk, unpack, 0)
    plsc.subcore_barrier()

    src_v = pk_v

    def gather_start(c, b):
      pltpu.async_copy(ht_hbm.at[src_v.at[c]], gbuf.at[b], gsems[b])

    def gather_wait(c, b):
      pltpu.make_async_copy(ht_hbm.at[src_v.at[c]], gbuf.at[b],
                            gsems[b]).wait()

    def scatter_start(c, b):
      pltpu.async_copy(sbuf.at[b], acc.at[dst_v.at[c]], ssems[b], add=True)

    def scatter_wait(c, b):
      pltpu.make_async_copy(sbuf.at[b], acc.at[dst_v.at[c]],
                            ssems[b]).wait()

    for b in range(NBUF):
      gather_start(b, b)

    def outer(g, _):
      for b in range(NBUF):
        c = g * NBUF + b
        gather_wait(c, b)

        @pl.when(g > 0)
        def _drain_prev_scatter():
          scatter_wait(c - NBUF, b)

        for gg in range(K // 16):
          w16 = w_v[pl.ds(c * K + gg * 16, 16)]
          for i in range(16):
            e = gg * 16 + i
            wv = lax.gather(
                w16, jnp.full((16, 1), i, jnp.int32),
                dimension_numbers=lax.GatherDimensionNumbers(
                    offset_dims=(), collapsed_slice_dims=(0,),
                    start_index_map=(0,)),
                slice_sizes=(1,),
                mode=lax.GatherScatterMode.PROMISE_IN_BOUNDS)
            for j in range(D // 16):
              sl = pl.ds(16 * j, 16)
              sbuf[b, e, sl] = gbuf[b, e, sl] * wv
        scatter_start(c, b)

        @pl.when(c + NBUF < nchunk)
        def _next_gather():
          gather_start(c + NBUF, b)
      return 0
    lax.fori_loop(0, nchunk // NBUF, outer, 0)
    for b in range(NBUF):
      scatter_wait(nchunk - NBUF + b, b)

    plsc.subcore_barrier()
    pltpu.sync_copy(acc.at[pl.ds(base, RPS)],
                    out_hbm.at[cid, pl.ds(base, RPS)])

    @pl.when(sid == NSUB - 1)
    def _copy_tail():
      pltpu.sync_copy(acc.at[pl.ds(NSUB * RPS, TAIL)],
                      out_hbm.at[cid, pl.ds(NSUB * RPS, TAIL)])

  return pl.kernel(
      body,
      out_type=jax.ShapeDtypeStruct((NCORES, N, D), jnp.float32),
      mesh=mesh,
      compiler_params=pltpu.CompilerParams(use_tc_tiling_on_sc=False),
      scratch_types=[
          pltpu.VMEM((nchunk, K), jnp.int32),
          pltpu.VMEM((nchunk, K), jnp.int32),
          pltpu.VMEM((epw,), jnp.float32),
          pltpu.VMEM((NBUF, K, D), jnp.float32),
          pltpu.VMEM((NBUF, K, D), jnp.float32),
          pltpu.VMEM((ZR, D), jnp.float32),
          pltpu.VMEM_SHARED((N, D), jnp.float32),
          [pltpu.SemaphoreType.DMA] * NBUF,
          [pltpu.SemaphoreType.DMA] * NBUF,
      ],
  )


_agg_col = _make_agg(H, True)
_agg_col3 = _make_agg(D3 // 2, True)

_R = 1000


def _mm_split_body(x_ref, w_ref, o_ref):
  res = jnp.dot(x_ref[...], w_ref[...], preferred_element_type=jnp.float32)
  o_ref[0] = res[:, :H]
  o_ref[1] = res[:, H:]


def _mm_split(x, w):
  n, f = x.shape
  return pl.pallas_call(
      _mm_split_body,
      grid=(n // _R,),
      in_specs=[pl.BlockSpec((_R, f), lambda i: (i, 0)),
                pl.BlockSpec((f, 2 * H), lambda i: (0, 0))],
      out_specs=pl.BlockSpec((2, _R, H), lambda i: (0, i, 0)),
      out_shape=jax.ShapeDtypeStruct((2, n, H), jnp.float32),
  )(x, w)


def _bn_relu_mm_body(p_ref, g_ref, b_ref, w_ref, o_ref):
  h0 = jnp.maximum(p_ref[0] * (g_ref[0] * BN_RS) + b_ref[0], 0.0)
  h1 = jnp.maximum(p_ref[1] * (g_ref[1] * BN_RS) + b_ref[1], 0.0)
  res = (jnp.dot(h0, w_ref[0], preferred_element_type=jnp.float32) +
         jnp.dot(h1, w_ref[1], preferred_element_type=jnp.float32))
  half = res.shape[1] // 2
  o_ref[0] = res[:, :half]
  o_ref[1] = res[:, half:]


def _bn_relu_mm(p, gamma, beta, w):
  n = p.shape[1]
  co = w.shape[1]
  g2 = gamma.reshape(2, H)
  b2 = beta.reshape(2, H)
  w3 = w.reshape(2, H, co)
  return pl.pallas_call(
      _bn_relu_mm_body,
      grid=(n // _R,),
      in_specs=[pl.BlockSpec((2, _R, H), lambda i: (0, i, 0)),
                pl.BlockSpec((2, H), lambda i: (0, 0)),
                pl.BlockSpec((2, H), lambda i: (0, 0)),
                pl.BlockSpec((2, H, co), lambda i: (0, 0, 0))],
      out_specs=pl.BlockSpec((2, _R, co // 2), lambda i: (0, i, 0)),
      out_shape=jax.ShapeDtypeStruct((2, n, co // 2), jnp.float32),
  )(p, g2, b2, w3)


def _bias_body(p_ref, b_ref, o_ref):
  s = jnp.concatenate([p_ref[0], p_ref[1]], axis=1)
  o_ref[...] = s[:, :NC] + b_ref[...]


def _bias_cat(p, b3):
  n = p.shape[1]
  b2 = b3.reshape(1, NC)
  return pl.pallas_call(
      _bias_body,
      grid=(n // _R,),
      in_specs=[pl.BlockSpec((2, _R, D3 // 2), lambda i: (0, i, 0)),
                pl.BlockSpec((1, NC), lambda i: (0, 0))],
      out_specs=pl.BlockSpec((_R, NC), lambda i: (i, 0)),
      out_shape=jax.ShapeDtypeStruct((n, NC), jnp.float32),
  )(p, b2)


@jax.jit
def kernel(feat, edge_index, edge_weight, W1, gamma1, beta1,
           W2, gamma2, beta2, W3, b3):
  pk = (edge_index[0] << 16) | edge_index[1]
  pk_c = pk.reshape(NSUB, E // NSUB // K, K)
  w_c = edge_weight.reshape(NSUB, E // NSUB)
  W3p = jnp.pad(W3, ((0, 0), (0, D3 - NC)))

  ht1 = _mm_split(feat, W1).reshape(2 * N, H)
  p1 = _agg_col(ht1, pk_c, w_c)
  ht2 = _bn_relu_mm(p1, gamma1, beta1, W2).reshape(2 * N, H)
  p2 = _agg_col(ht2, pk_c, w_c)
  ht3 = _bn_relu_mm(p2, gamma2, beta2, W3p).reshape(2 * N, D3 // 2)
  p3 = _agg_col3(ht3, pk_c, w_c)
  return _bias_cat(p3, b3)
